# baseline (speedup 1.0000x reference)
import jax
import jax.numpy as jnp
from jax import lax
from jax.experimental import pallas as pl
from jax.experimental.pallas import tpu as pltpu

KTOP = 10
PAD = 16
W = 1024

_NEG_INF = float("-inf")
_PAD_ID = 2.0e9


def _insert(rv_ref, ri_ref, m, i, nq):
    rv = rv_ref[...]
    ri = ri_ref[...]
    rvs = jnp.concatenate(
        [jnp.full((nq, 1), jnp.inf, jnp.float32), rv[:, :-1]], axis=1)
    ris = jnp.concatenate([ri[:, :1], ri[:, :-1]], axis=1)
    ge = rv >= m
    gp = rvs >= m
    mb = jnp.broadcast_to(m, (nq, PAD))
    ib = jnp.broadcast_to(i, (nq, PAD))
    rv_ref[...] = jnp.where(ge, rv, jnp.where(gp, mb, rvs))
    ri_ref[...] = jnp.where(ge, ri, jnp.where(gp, ib, ris))


def _make_body(nq, ndb, nblocks):
    def body(q_ref, db_ref, idx_ref, sim_ref, rv_ref, ri_ref, t_ref, m_ref):
        j = pl.program_id(0)

        @pl.when(j == 0)
        def _():
            rv_ref[...] = jnp.full((nq, PAD), _NEG_INF, jnp.float32)
            ri_ref[...] = jnp.full((nq, PAD), _PAD_ID, jnp.float32)

        sims = lax.dot_general(
            q_ref[...], db_ref[...], (((1,), (1,)), ((), ())),
            preferred_element_type=jnp.float32,
        )
        sim_ref[...] = sims

        base = jnp.float32(j) * W

        @pl.when(j < nblocks - 1)
        def _():
            t_ref[...] = sims

        @pl.when(j == nblocks - 1)
        def _():
            ids0 = lax.broadcasted_iota(
                jnp.int32, (nq, W), 1).astype(jnp.float32)
            t_ref[...] = jnp.where(base + ids0 < ndb, sims, _NEG_INF)

        tau = rv_ref[:, KTOP - 1:KTOP]
        m_ref[...] = jnp.max(t_ref[...], axis=1, keepdims=True)

        for t in range(KTOP):
            pred = jnp.max(m_ref[...] - tau) > 0.0

            @pl.when(pred)
            def _():
                tl = t_ref[...]
                m = m_ref[...]
                idsf = base + lax.broadcasted_iota(
                    jnp.int32, (nq, W), 1).astype(jnp.float32)
                i = -jnp.max(jnp.where(tl == m, -idsf, _NEG_INF), axis=1,
                             keepdims=True)
                tl2 = jnp.where(idsf == i, _NEG_INF, tl)
                t_ref[...] = tl2
                m_ref[...] = jnp.max(tl2, axis=1, keepdims=True)
                _insert(rv_ref, ri_ref, m, i, nq)

        @pl.when(j == nblocks - 1)
        def _():
            idx_ref[...] = ri_ref[:, :KTOP].astype(jnp.int32)

    return body


def _l2norm(x):
    n = jnp.linalg.norm(x, ord=2, axis=1, keepdims=True)
    return x / jnp.maximum(n, 1e-12)


def kernel(vectors, db_vectors, k):
    nq, d = vectors.shape
    ndb = db_vectors.shape[0]
    nblocks = pl.cdiv(ndb, W)
    vectors = _l2norm(vectors)
    db_vectors = _l2norm(db_vectors)

    indices, sims = pl.pallas_call(
        _make_body(nq, ndb, nblocks),
        grid=(nblocks,),
        in_specs=[
            pl.BlockSpec((nq, d), lambda j: (0, 0)),
            pl.BlockSpec((W, d), lambda j: (j, 0)),
        ],
        out_specs=[
            pl.BlockSpec((nq, KTOP), lambda j: (0, 0)),
            pl.BlockSpec((nq, W), lambda j: (0, j)),
        ],
        out_shape=[
            jax.ShapeDtypeStruct((nq, KTOP), jnp.int32),
            jax.ShapeDtypeStruct((nq, ndb), jnp.float32),
        ],
        scratch_shapes=[
            pltpu.VMEM((nq, PAD), jnp.float32),
            pltpu.VMEM((nq, PAD), jnp.float32),
            pltpu.VMEM((nq, W), jnp.float32),
            pltpu.VMEM((nq, 1), jnp.float32),
        ],
    )(vectors, db_vectors)
    return indices, sims


# speedup vs baseline: 1.2951x; 1.2951x over previous
import jax
import jax.numpy as jnp
from jax import lax
from jax.experimental import pallas as pl
from jax.experimental.pallas import tpu as pltpu

KTOP = 10
PAD = 16
W = 1024

_NEG_INF = float("-inf")
_PAD_ID = 2.0e9


def _insert(rv_ref, ri_ref, m, i, nq):
    rv = rv_ref[...]
    ri = ri_ref[...]
    rvs = jnp.concatenate(
        [jnp.full((nq, 1), jnp.inf, jnp.float32), rv[:, :-1]], axis=1)
    ris = jnp.concatenate([ri[:, :1], ri[:, :-1]], axis=1)
    ge = rv >= m
    gp = rvs >= m
    mb = jnp.broadcast_to(m, (nq, PAD))
    ib = jnp.broadcast_to(i, (nq, PAD))
    rv_ref[...] = jnp.where(ge, rv, jnp.where(gp, mb, rvs))
    ri_ref[...] = jnp.where(ge, ri, jnp.where(gp, ib, ris))


def _make_body(nq, ndb, nblocks):
    def body(q_ref, db_ref, idx_ref, sim_ref, rv_ref, ri_ref, t_ref):
        j = pl.program_id(0)

        @pl.when(j == 0)
        def _():
            rv_ref[...] = jnp.full((nq, PAD), _NEG_INF, jnp.float32)
            ri_ref[...] = jnp.full((nq, PAD), _PAD_ID, jnp.float32)

        sims = lax.dot_general(
            q_ref[...], db_ref[...], (((1,), (1,)), ((), ())),
            preferred_element_type=jnp.float32,
        )
        sim_ref[...] = sims

        base = jnp.float32(j) * W
        ids0 = lax.broadcasted_iota(jnp.int32, (nq, W), 1).astype(jnp.float32)
        t_ref[...] = jnp.where(base + ids0 < ndb, sims, _NEG_INF)

        tau = rv_ref[:, KTOP - 1:KTOP]
        cnt = jnp.sum((t_ref[...] > tau).astype(jnp.float32), axis=1,
                      keepdims=True)
        n_iter = jnp.max(cnt)

        for t in range(KTOP):
            @pl.when(n_iter > jnp.float32(t))
            def _():
                tl = t_ref[...]
                m = jnp.max(tl, axis=1, keepdims=True)
                idsf = base + lax.broadcasted_iota(
                    jnp.int32, (nq, W), 1).astype(jnp.float32)
                i = -jnp.max(jnp.where(tl == m, -idsf, _NEG_INF), axis=1,
                             keepdims=True)
                t_ref[...] = jnp.where(idsf == i, _NEG_INF, tl)
                _insert(rv_ref, ri_ref, m, i, nq)

        @pl.when(j == nblocks - 1)
        def _():
            idx_ref[...] = ri_ref[:, :KTOP].astype(jnp.int32)

    return body


def _l2norm(x):
    n = jnp.linalg.norm(x, ord=2, axis=1, keepdims=True)
    return x / jnp.maximum(n, 1e-12)


def kernel(vectors, db_vectors, k):
    nq, d = vectors.shape
    ndb = db_vectors.shape[0]
    nblocks = pl.cdiv(ndb, W)
    vectors = _l2norm(vectors)
    db_vectors = _l2norm(db_vectors)

    indices, sims = pl.pallas_call(
        _make_body(nq, ndb, nblocks),
        grid=(nblocks,),
        in_specs=[
            pl.BlockSpec((nq, d), lambda j: (0, 0)),
            pl.BlockSpec((W, d), lambda j: (j, 0)),
        ],
        out_specs=[
            pl.BlockSpec((nq, KTOP), lambda j: (0, 0)),
            pl.BlockSpec((nq, W), lambda j: (0, j)),
        ],
        out_shape=[
            jax.ShapeDtypeStruct((nq, KTOP), jnp.int32),
            jax.ShapeDtypeStruct((nq, ndb), jnp.float32),
        ],
        scratch_shapes=[
            pltpu.VMEM((nq, PAD), jnp.float32),
            pltpu.VMEM((nq, PAD), jnp.float32),
            pltpu.VMEM((nq, W), jnp.float32),
        ],
    )(vectors, db_vectors)
    return indices, sims
